# baseline (speedup 1.0000x reference)
import functools

import jax
import jax.numpy as jnp
from jax import lax
from jax.experimental import pallas as pl
from jax.experimental.pallas import tpu as pltpu
from jax.experimental.pallas import tpu_sc as plsc

NC = 2
NS = 16
NW = NC * NS
C = 128
G = 64


@functools.lru_cache(maxsize=None)
def _sc_scatter(n_pad, nc0, nc1, h, ones_mode=False, wout=None):
  if wout is None:
    wout = h
  stripe = n_pad // NS
  assert stripe % 64 == 0 and n_pad % 8 == 0
  assert ones_mode or (nc0 % 2 == 0 and nc1 % 2 == 0)
  ncmax = max(nc0, nc1)
  mesh = plsc.VectorSubcoreMesh(
      core_axis_name="c", subcore_axis_name="s", num_cores=NC,
      num_subcores=NS)

  def body(table, src2, dst2, out, sidx, didx, rows0, rows1, zbuf, acc,
           gsem):
    core = lax.axis_index("c")
    sid = lax.axis_index("s")

    for r in range(64):
      for q in range(h // 16):
        zbuf[r, q * 16:(q + 1) * 16] = jnp.zeros((16,), jnp.float32)

    @pl.loop(0, stripe // 64)
    def _(i):
      pltpu.sync_copy(zbuf, acc.at[pl.ds(sid * stripe + i * 64, 64)])

    plsc.subcore_barrier()

    if ones_mode:
      for r in range(C):
        for q in range(h // 16):
          rows0[r, q * 16:(q + 1) * 16] = jnp.full((16,), 1.0, jnp.float32)

    def run(nc, base):
      if ones_mode:
        pltpu.sync_copy(dst2.at[pl.ds(base, nc)], didx.at[pl.ds(0, nc)])

        @pl.loop(0, nc)
        def _(j):
          pltpu.sync_copy(rows0, acc.at[didx.at[j]], add=True)
      else:
        pltpu.sync_copy(src2.at[pl.ds(base, nc)], sidx.at[pl.ds(0, nc)])
        pltpu.sync_copy(dst2.at[pl.ds(base, nc)], didx.at[pl.ds(0, nc)])
        bufs = (rows0, rows1)
        pltpu.async_copy(table.at[sidx.at[0]], rows0, gsem)

        @pl.loop(0, nc, step=2)
        def _(j):
          for b in range(2):
            cur = j + b
            pltpu.make_async_copy(table.at[sidx.at[cur]], bufs[b],
                                  gsem).wait()

            @pl.when(cur + 1 < nc)
            def _():
              pltpu.async_copy(table.at[sidx.at[cur + 1]], bufs[1 - b], gsem)

            pltpu.sync_copy(bufs[b], acc.at[didx.at[cur]], add=True)

    @pl.when(core == 0)
    def _():
      run(nc0, sid * nc0)

    @pl.when(core == 1)
    def _():
      run(nc1, NS * nc0 + sid * nc1)

    plsc.subcore_barrier()
    off = core * n_pad + sid * stripe
    if wout == h:
      dst = out.at[pl.ds(off, stripe)]
    else:
      dst = out.at[pl.ds(off, stripe), pl.ds(0, h)]
    pltpu.sync_copy(acc.at[pl.ds(sid * stripe, stripe)], dst)

  return pl.kernel(
      body,
      out_type=jax.ShapeDtypeStruct((NC * n_pad, wout), jnp.float32),
      mesh=mesh,
      compiler_params=pltpu.CompilerParams(use_tc_tiling_on_sc=False),
      scratch_types=[
          pltpu.VMEM((ncmax, C), jnp.int32),
          pltpu.VMEM((ncmax, C), jnp.int32),
          pltpu.VMEM((C, h), jnp.float32),
          pltpu.VMEM((C, h), jnp.float32),
          pltpu.VMEM((64, h), jnp.float32),
          pltpu.VMEM_SHARED((n_pad, h), jnp.float32),
          pltpu.SemaphoreType.DMA,
      ],
  )


_PREC = jax.lax.Precision.DEFAULT


def _dot(a, b):
  return jax.lax.dot_general(a, b, (((1,), (0,)), ((), ())),
                             precision=_PREC,
                             preferred_element_type=jnp.float32)


def _tc_matmul(x_pad, w1):
  n_pad = x_pad.shape[0]
  h = w1.shape[1]

  def body(x_ref, w_ref, m_ref):
    m_ref[...] = _dot(x_ref[...], w_ref[...])

  return pl.pallas_call(
      body,
      out_shape=jax.ShapeDtypeStruct((n_pad, h), jnp.float32),
  )(x_pad, w1)


def _tc_stage1(sdeg, m1):
  n_pad = m1.shape[0]
  h = m1.shape[1]

  def body(sdeg_ref, m_ref, g_ref, dinv_ref):
    deg = sdeg_ref[:n_pad, 0:1] + sdeg_ref[n_pad:, 0:1] + 1.0
    dinv = jax.lax.rsqrt(deg)
    g_ref[...] = m_ref[...] * dinv
    dinv_ref[...] = dinv

  return pl.pallas_call(
      body,
      out_shape=(jax.ShapeDtypeStruct((n_pad, h), jnp.float32),
                 jax.ShapeDtypeStruct((n_pad, 1), jnp.float32)),
  )(sdeg, m1)


def _tc_mid(s, g_prev, dinv, b, w, relu):
  n_pad = g_prev.shape[0]
  h_dim = w.shape[1]
  blk = 1024
  nb = n_pad // blk
  ws = s.shape[1]

  def body(s0_ref, s1_ref, g_ref, dinv_ref, b_ref, w_ref, out_ref):
    dinv = dinv_ref[...]
    hid = (dinv * (s0_ref[:, :h_dim] + s1_ref[:, :h_dim] + g_ref[...])
           + b_ref[...])
    if relu:
      hid = jnp.maximum(hid, 0.0)
    out_ref[...] = _dot(hid, w_ref[...]) * dinv

  return pl.pallas_call(
      body,
      grid=(nb,),
      in_specs=[
          pl.BlockSpec((blk, ws), lambda i: (i, 0)),
          pl.BlockSpec((blk, ws), lambda i, _nb=nb: (i + _nb, 0)),
          pl.BlockSpec((blk, h_dim), lambda i: (i, 0)),
          pl.BlockSpec((blk, 1), lambda i: (i, 0)),
          pl.BlockSpec((1, h_dim), lambda i: (0, 0)),
          pl.BlockSpec((h_dim, h_dim), lambda i: (0, 0)),
      ],
      out_specs=pl.BlockSpec((blk, h_dim), lambda i: (i, 0)),
      out_shape=jax.ShapeDtypeStruct((n_pad, h_dim), jnp.float32),
  )(s, s, g_prev, dinv, b, w)


def _tc_final(s, g3, dinv, b3, batch_row, wp, bp, n_pad):
  def body(s_ref, g_ref, dinv_ref, b_ref, batch_ref, wp_ref, bp_ref,
           out_ref):
    hd = g_ref.shape[1]
    hid = dinv_ref[...] * (s_ref[:n_pad, :hd] + s_ref[n_pad:, :hd]
                           + g_ref[...])
    hid = hid + b_ref[...]
    iota = jax.lax.broadcasted_iota(jnp.int32, (G, n_pad), 0)
    oh = (iota == batch_ref[...]).astype(jnp.float32)
    sums = _dot(oh, hid)
    cnt = _dot(oh, jnp.ones((n_pad, 1), jnp.float32))
    emb = sums / jnp.maximum(cnt, 1.0)
    out_ref[...] = _dot(emb, wp_ref[...]) + bp_ref[...]

  return pl.pallas_call(
      body,
      out_shape=jax.ShapeDtypeStruct((G, 1), jnp.float32),
  )(s, g3, dinv, b3, batch_row, wp, bp)


def kernel(x, edge_index, batch, W1, b1, W2, b2, W3, b3, Wp, bp):
  n, d = x.shape
  h = W1.shape[1]
  e = edge_index.shape[1]

  n_pad = ((n + 1 + 1023) // 1024) * 1024
  t = -(-e // C)
  nc_sum = -(-t // NS)
  nc0 = 2 * int(round(0.5 * nc_sum / 2.0))
  nc1 = nc_sum - nc0
  nc1 += nc1 % 2
  t_pad = NS * (nc0 + nc1)
  e_pad = t_pad * C

  src = edge_index[0].astype(jnp.int32)
  dst = edge_index[1].astype(jnp.int32)
  npad_ids = jnp.arange(e_pad - e, dtype=jnp.int32)
  src_pad = npad_ids % n
  dst_pad = n + npad_ids % (n_pad - n)
  src2 = jnp.concatenate([src, src_pad]).reshape(t_pad, C)
  dst2 = jnp.concatenate([dst, dst_pad]).reshape(t_pad, C)

  x_pad = jnp.concatenate(
      [x, jnp.zeros((n_pad - n, d), jnp.float32)], axis=0)
  batch_row = jnp.concatenate(
      [batch.astype(jnp.int32),
       jnp.full((n_pad - n,), G, dtype=jnp.int32)]).reshape(1, n_pad)

  ones16 = jnp.ones((8, 16), jnp.float32)
  nd = (nc0 + nc1) // 2
  sdeg = _sc_scatter(n_pad, nd, nc0 + nc1 - nd, 16,
                     ones_mode=True)(ones16, src2, dst2)

  m1 = _tc_matmul(x_pad, W1)
  g1, dinv = _tc_stage1(sdeg, m1)

  prop = _sc_scatter(n_pad, nc0, nc1, h, wout=128)
  s1 = prop(g1, src2, dst2)
  g2 = _tc_mid(s1, g1, dinv, b1.reshape(1, h), W2, relu=True)
  s2 = prop(g2, src2, dst2)
  g3 = _tc_mid(s2, g2, dinv, b2.reshape(1, h), W3, relu=True)
  s3 = prop(g3, src2, dst2)
  return _tc_final(s3, g3, dinv, b3.reshape(1, h), batch_row,
                   Wp, bp.reshape(1, 1), n_pad)


# speedup vs baseline: 1.0117x; 1.0117x over previous
import functools

import jax
import jax.numpy as jnp
from jax import lax
from jax.experimental import pallas as pl
from jax.experimental.pallas import tpu as pltpu
from jax.experimental.pallas import tpu_sc as plsc

NC = 2
NS = 16
NW = NC * NS
C = 128
G = 64


@functools.lru_cache(maxsize=None)
def _sc_scatter(n_pad, nc0, nc1, h, ones_mode=False, wout=None):
  if wout is None:
    wout = h
  stripe = n_pad // NS
  assert stripe % 64 == 0 and n_pad % 8 == 0
  assert ones_mode or (nc0 % 2 == 0 and nc1 % 2 == 0)
  ncmax = max(nc0, nc1)
  mesh = plsc.VectorSubcoreMesh(
      core_axis_name="c", subcore_axis_name="s", num_cores=NC,
      num_subcores=NS)

  def body(table, src2, dst2, out, sidx, didx, rows0, rows1, zbuf, acc,
           gsem):
    core = lax.axis_index("c")
    sid = lax.axis_index("s")

    for r in range(64):
      for q in range(h // 16):
        zbuf[r, q * 16:(q + 1) * 16] = jnp.zeros((16,), jnp.float32)

    @pl.loop(0, stripe // 64)
    def _(i):
      pltpu.sync_copy(zbuf, acc.at[pl.ds(sid * stripe + i * 64, 64)])

    plsc.subcore_barrier()

    if ones_mode:
      for r in range(C):
        for q in range(h // 16):
          rows0[r, q * 16:(q + 1) * 16] = jnp.full((16,), 1.0, jnp.float32)

    def run(nc, base):
      if ones_mode:
        pltpu.sync_copy(dst2.at[pl.ds(base, nc)], didx.at[pl.ds(0, nc)])

        @pl.loop(0, nc)
        def _(j):
          pltpu.sync_copy(rows0, acc.at[didx.at[j]], add=True)
      else:
        pltpu.sync_copy(src2.at[pl.ds(base, nc)], sidx.at[pl.ds(0, nc)])
        pltpu.sync_copy(dst2.at[pl.ds(base, nc)], didx.at[pl.ds(0, nc)])
        bufs = (rows0, rows1)
        pltpu.async_copy(table.at[sidx.at[0]], rows0, gsem)

        @pl.loop(0, nc, step=2)
        def _(j):
          for b in range(2):
            cur = j + b
            pltpu.make_async_copy(table.at[sidx.at[cur]], bufs[b],
                                  gsem).wait()

            @pl.when(cur + 1 < nc)
            def _():
              pltpu.async_copy(table.at[sidx.at[cur + 1]], bufs[1 - b], gsem)

            pltpu.sync_copy(bufs[b], acc.at[didx.at[cur]], add=True)

    @pl.when(core == 0)
    def _():
      run(nc0, sid * nc0)

    @pl.when(core == 1)
    def _():
      run(nc1, NS * nc0 + sid * nc1)

    plsc.subcore_barrier()
    off = core * n_pad + sid * stripe
    if wout == h:
      dst = out.at[pl.ds(off, stripe)]
    else:
      dst = out.at[pl.ds(off, stripe), pl.ds(0, h)]
    pltpu.sync_copy(acc.at[pl.ds(sid * stripe, stripe)], dst)

  return pl.kernel(
      body,
      out_type=jax.ShapeDtypeStruct((NC * n_pad, wout), jnp.float32),
      mesh=mesh,
      compiler_params=pltpu.CompilerParams(use_tc_tiling_on_sc=False),
      scratch_types=[
          pltpu.VMEM((ncmax, C), jnp.int32),
          pltpu.VMEM((ncmax, C), jnp.int32),
          pltpu.VMEM((C, h), jnp.float32),
          pltpu.VMEM((C, h), jnp.float32),
          pltpu.VMEM((64, h), jnp.float32),
          pltpu.VMEM_SHARED((n_pad, h), jnp.float32),
          pltpu.SemaphoreType.DMA,
      ],
  )


_PREC = jax.lax.Precision.DEFAULT


def _dot(a, b):
  return jax.lax.dot_general(a, b, (((1,), (0,)), ((), ())),
                             precision=_PREC,
                             preferred_element_type=jnp.float32)


def _tc_stage1(sdeg, x_pad, w1):
  n_pad = x_pad.shape[0]
  h = w1.shape[1]

  def body(sdeg_ref, x_ref, w_ref, g_ref, dinv_ref):
    deg = sdeg_ref[:n_pad, 0:1] + sdeg_ref[n_pad:, 0:1] + 1.0
    dinv = jax.lax.rsqrt(deg)
    m = _dot(x_ref[...], w_ref[...])
    g_ref[...] = m * dinv
    dinv_ref[...] = dinv

  return pl.pallas_call(
      body,
      out_shape=(jax.ShapeDtypeStruct((n_pad, h), jnp.float32),
                 jax.ShapeDtypeStruct((n_pad, 1), jnp.float32)),
  )(sdeg, x_pad, w1)


def _tc_mid(s, g_prev, dinv, b, w, relu):
  n_pad = g_prev.shape[0]
  h_dim = w.shape[1]

  def body(s_ref, g_ref, dinv_ref, b_ref, w_ref, out_ref):
    dinv = dinv_ref[...]
    hid = (dinv * (s_ref[:n_pad, :h_dim] + s_ref[n_pad:, :h_dim]
                   + g_ref[...]) + b_ref[...])
    if relu:
      hid = jnp.maximum(hid, 0.0)
    out_ref[...] = _dot(hid, w_ref[...]) * dinv

  return pl.pallas_call(
      body,
      out_shape=jax.ShapeDtypeStruct((n_pad, h_dim), jnp.float32),
  )(s, g_prev, dinv, b, w)


def _tc_final(s, g3, dinv, b3, batch_row, wp, bp, n_pad):
  def body(s_ref, g_ref, dinv_ref, b_ref, batch_ref, wp_ref, bp_ref,
           out_ref):
    hd = g_ref.shape[1]
    hid = dinv_ref[...] * (s_ref[:n_pad, :hd] + s_ref[n_pad:, :hd]
                           + g_ref[...])
    hid = hid + b_ref[...]
    iota = jax.lax.broadcasted_iota(jnp.int32, (G, n_pad), 0)
    oh = (iota == batch_ref[...]).astype(jnp.float32)
    sums = _dot(oh, hid)
    cnt = _dot(oh, jnp.ones((n_pad, 1), jnp.float32))
    emb = sums / jnp.maximum(cnt, 1.0)
    out_ref[...] = _dot(emb, wp_ref[...]) + bp_ref[...]

  return pl.pallas_call(
      body,
      out_shape=jax.ShapeDtypeStruct((G, 1), jnp.float32),
  )(s, g3, dinv, b3, batch_row, wp, bp)


def kernel(x, edge_index, batch, W1, b1, W2, b2, W3, b3, Wp, bp):
  n, d = x.shape
  h = W1.shape[1]
  e = edge_index.shape[1]

  n_pad = ((n + 1 + 1023) // 1024) * 1024
  t = -(-e // C)
  nc_sum = -(-t // NS)
  nc0 = 2 * int(round(0.5 * nc_sum / 2.0))
  nc1 = nc_sum - nc0
  nc1 += nc1 % 2
  t_pad = NS * (nc0 + nc1)
  e_pad = t_pad * C

  src = edge_index[0].astype(jnp.int32)
  dst = edge_index[1].astype(jnp.int32)
  npad_ids = jnp.arange(e_pad - e, dtype=jnp.int32)
  src_pad = npad_ids % n
  dst_pad = n + npad_ids % (n_pad - n)
  src2 = jnp.concatenate([src, src_pad]).reshape(t_pad, C)
  dst2 = jnp.concatenate([dst, dst_pad]).reshape(t_pad, C)

  x_pad = jnp.concatenate(
      [x, jnp.zeros((n_pad - n, d), jnp.float32)], axis=0)
  batch_row = jnp.concatenate(
      [batch.astype(jnp.int32),
       jnp.full((n_pad - n,), G, dtype=jnp.int32)]).reshape(1, n_pad)

  ones16 = jnp.ones((8, 16), jnp.float32)
  nd = (nc0 + nc1) // 2
  sdeg = _sc_scatter(n_pad, nd, nc0 + nc1 - nd, 16,
                     ones_mode=True)(ones16, src2, dst2)

  g1, dinv = _tc_stage1(sdeg, x_pad, W1)

  prop = _sc_scatter(n_pad, nc0, nc1, h, wout=128)
  s1 = prop(g1, src2, dst2)
  g2 = _tc_mid(s1, g1, dinv, b1.reshape(1, h), W2, relu=True)
  s2 = prop(g2, src2, dst2)
  g3 = _tc_mid(s2, g2, dinv, b2.reshape(1, h), W3, relu=True)
  s3 = prop(g3, src2, dst2)
  return _tc_final(s3, g3, dinv, b3.reshape(1, h), batch_row,
                   Wp, bp.reshape(1, 1), n_pad)
